# Initial kernel scaffold; baseline (speedup 1.0000x reference)
import functools

import jax
import jax.numpy as jnp
from jax import lax
from jax.experimental import pallas as pl
from jax.experimental.pallas import tpu as pltpu
from jax.experimental.pallas import tpu_sc as plsc

N = 10000
E = 160000
D = 256
H = D // 2
NC = 2
NS = 16
CHUNK = 128

NPAD = 10240
EPAD = 163840
ROWS_PER_TILE = NPAD // NS
SEG_CHUNKS = EPAD // (NS * CHUNK)
DEG_CHUNKS = EPAD // (NC * NS * CHUNK)

_mesh = plsc.VectorSubcoreMesh(core_axis_name="c", subcore_axis_name="s")


@functools.partial(
    pl.kernel,
    out_type=jax.ShapeDtypeStruct((NC, NPAD, 16), jnp.float32),
    mesh=_mesh,
    scratch_types=[
        pltpu.VMEM((DEG_CHUNKS, CHUNK), jnp.int32),
        pltpu.VMEM((CHUNK, 16), jnp.float32),
        pltpu.VMEM((ROWS_PER_TILE, 16), jnp.float32),
        pltpu.VMEM_SHARED((NPAD, 16), jnp.float32),
    ],
)
def _deg_kernel(dst_hbm, out_hbm, idx_v, ones_v, stage_v, acc):
    c = lax.axis_index("c")
    s = lax.axis_index("s")
    w = c * NS + s

    def fill(i, _):
        ones_v[i, :] = jnp.full((16,), 1.0, jnp.float32)
        return 0
    lax.fori_loop(0, CHUNK, fill, 0)

    def zfill(i, _):
        stage_v[i, :] = jnp.zeros((16,), jnp.float32)
        return 0
    lax.fori_loop(0, ROWS_PER_TILE, zfill, 0)
    pltpu.sync_copy(stage_v, acc.at[pl.ds(s * ROWS_PER_TILE, ROWS_PER_TILE)])
    plsc.subcore_barrier()

    pltpu.sync_copy(dst_hbm.at[w], idx_v)

    def body(j, _):
        pltpu.sync_copy(ones_v, acc.at[idx_v.at[j]], add=True)
        return 0
    lax.fori_loop(0, DEG_CHUNKS, body, 0)
    plsc.subcore_barrier()

    sl = pl.ds(s * ROWS_PER_TILE, ROWS_PER_TILE)
    pltpu.sync_copy(acc.at[sl], stage_v)
    pltpu.sync_copy(stage_v, out_hbm.at[c].at[sl])


@functools.partial(
    pl.kernel,
    out_type=jax.ShapeDtypeStruct((NC, NPAD, H), jnp.float32),
    mesh=_mesh,
    scratch_types=[
        pltpu.VMEM((SEG_CHUNKS, CHUNK), jnp.int32),
        pltpu.VMEM((SEG_CHUNKS, CHUNK), jnp.int32),
        pltpu.VMEM((CHUNK, H), jnp.float32),
        pltpu.VMEM((CHUNK, H), jnp.float32),
        pltpu.VMEM_SHARED((NPAD, H), jnp.float32),
        pltpu.SemaphoreType.DMA,
    ],
)
def _segsum_kernel(g_hbm, src_hbm, dst_hbm, out_hbm,
                   src_v, dst_v, rows_v, stage_v, acc, sem):
    c = lax.axis_index("c")
    s = lax.axis_index("s")

    def zfill(i, _):
        for k in range(H // 16):
            stage_v[i, pl.ds(k * 16, 16)] = jnp.zeros((16,), jnp.float32)
        return 0
    lax.fori_loop(0, CHUNK, zfill, 0)
    for k in range(ROWS_PER_TILE // CHUNK):
        pltpu.sync_copy(
            stage_v, acc.at[pl.ds(s * ROWS_PER_TILE + k * CHUNK, CHUNK)])
    plsc.subcore_barrier()

    pltpu.sync_copy(src_hbm.at[s], src_v)
    pltpu.sync_copy(dst_hbm.at[s], dst_v)
    table = g_hbm.at[c]

    def body(j, _):
        pltpu.async_copy(table.at[src_v.at[j]], rows_v, sem).wait()
        pltpu.sync_copy(rows_v, acc.at[dst_v.at[j]], add=True)
        return 0
    lax.fori_loop(0, SEG_CHUNKS, body, 0)
    plsc.subcore_barrier()

    for k in range(ROWS_PER_TILE // CHUNK):
        sl = pl.ds(s * ROWS_PER_TILE + k * CHUNK, CHUNK)
        pltpu.sync_copy(acc.at[sl], stage_v)
        pltpu.sync_copy(stage_v, out_hbm.at[c].at[sl])


RB = 1000


def _dinv_of(deg2):
    return lax.rsqrt(jnp.sum(deg2, axis=(0, 2)) + 1.0)


def _tc1_body(deg2_ref, x_ref, w_ref, g_ref):
    dinv = _dinv_of(deg2_ref[...])
    h = jnp.dot(x_ref[...], w_ref[...], preferred_element_type=jnp.float32)
    g = h * dinv[:, None]
    g_ref[0] = g[:, :H]
    g_ref[1] = g[:, H:]


def _tc2_body(deg2_ref, s_ref, g_ref, w_ref, b_ref, o_ref):
    dinv = _dinv_of(deg2_ref[...])
    t = jnp.concatenate([s_ref[0] + g_ref[0], s_ref[1] + g_ref[1]], axis=1)
    z = jax.nn.relu(dinv[:, None] * t + b_ref[0][None, :])
    h = jnp.dot(z, w_ref[...], preferred_element_type=jnp.float32)
    g = h * dinv[:, None]
    o_ref[0] = g[:, :H]
    o_ref[1] = g[:, H:]


def _tc3_body(deg2_ref, s_ref, g_ref, b_ref, o_ref):
    dinv = _dinv_of(deg2_ref[...])
    t = jnp.concatenate([s_ref[0] + g_ref[0], s_ref[1] + g_ref[1]], axis=1)
    o_ref[...] = dinv[:, None] * t + b_ref[0][None, :]


_deg2_spec = pl.BlockSpec((NC, RB, 16), lambda i: (0, i, 0))
_half_spec = pl.BlockSpec((NC, RB, H), lambda i: (0, i, 0))
_b_spec = pl.BlockSpec((1, D), lambda i: (0, 0))
_w_spec = pl.BlockSpec((D, D), lambda i: (0, 0))

_tc1 = pl.pallas_call(
    _tc1_body,
    grid=(N // RB,),
    in_specs=[_deg2_spec, pl.BlockSpec((RB, D), lambda i: (i, 0)), _w_spec],
    out_specs=_half_spec,
    out_shape=jax.ShapeDtypeStruct((NC, N, H), jnp.float32),
)

_tc2 = pl.pallas_call(
    _tc2_body,
    grid=(N // RB,),
    in_specs=[_deg2_spec, _half_spec, _half_spec, _w_spec, _b_spec],
    out_specs=_half_spec,
    out_shape=jax.ShapeDtypeStruct((NC, N, H), jnp.float32),
)

_tc3 = pl.pallas_call(
    _tc3_body,
    grid=(N // RB,),
    in_specs=[_deg2_spec, _half_spec, _half_spec, _b_spec],
    out_specs=pl.BlockSpec((RB, D), lambda i: (i, 0)),
    out_shape=jax.ShapeDtypeStruct((N, D), jnp.float32),
)


def kernel(x, edge_index, W1, b1, W2, b2):
    src = edge_index[0].astype(jnp.int32)
    dst = edge_index[1].astype(jnp.int32)
    pad = EPAD - E
    src = jnp.concatenate([src, jnp.zeros((pad,), jnp.int32)])
    dst = jnp.concatenate([dst, jnp.full((pad,), N, jnp.int32)])
    dst_deg = dst.reshape(NC * NS, DEG_CHUNKS, CHUNK)
    src_seg = src.reshape(NS, SEG_CHUNKS, CHUNK)
    dst_seg = dst.reshape(NS, SEG_CHUNKS, CHUNK)

    deg2 = _deg_kernel(dst_deg)

    g1 = _tc1(deg2, x, W1)
    s1 = _segsum_kernel(g1, src_seg, dst_seg)
    g2 = _tc2(deg2, s1, g1, W2, jnp.reshape(b1, (1, D)))
    s2 = _segsum_kernel(g2, src_seg, dst_seg)
    return _tc3(deg2, s2, g2, jnp.reshape(b2, (1, D)))


# speedup vs baseline: 6.6907x; 6.6907x over previous
import functools

import jax
import jax.numpy as jnp
from jax import lax
from jax.experimental import pallas as pl
from jax.experimental.pallas import tpu as pltpu
from jax.experimental.pallas import tpu_sc as plsc

N = 10000
E = 160000
D = 256
H = D // 2
NC = 2
NS = 16
CHUNK = 128

NPAD = 10240
EPAD = 163840
ROWS_PER_TILE = NPAD // NS
SEG_CHUNKS = EPAD // (NS * CHUNK)
DEG_CHUNKS = EPAD // (NC * NS * CHUNK)

_mesh = plsc.VectorSubcoreMesh(core_axis_name="c", subcore_axis_name="s")


def _zero_stage(stage_v):
    def zfill(i, _):
        for q in range(128 // 16):
            stage_v[i, pl.ds(q * 16, 16)] = jnp.zeros((16,), jnp.float32)
        return 0
    lax.fori_loop(0, CHUNK, zfill, 0)


def _zero_acc_stripe(s, stage_v, acc):
    for t in range(NS):
        @pl.when(s == t)
        def _():
            for j in range(ROWS_PER_TILE // CHUNK):
                pltpu.sync_copy(
                    stage_v, acc.at[pl.ds(t * ROWS_PER_TILE + j * CHUNK, CHUNK)])


def _flush_acc_stripe(c, s, stage_v, acc, out_hbm):
    for t in range(NS):
        @pl.when(s == t)
        def _():
            for j in range(ROWS_PER_TILE // CHUNK):
                off = t * ROWS_PER_TILE + j * CHUNK
                pltpu.sync_copy(acc.at[pl.ds(off, CHUNK)], stage_v)
                pltpu.sync_copy(stage_v, out_hbm.at[pl.ds(c * NPAD + off, CHUNK)])


@functools.partial(
    pl.kernel,
    out_type=jax.ShapeDtypeStruct((NC * NPAD, 128), jnp.float32),
    mesh=_mesh,
    scratch_types=[
        pltpu.VMEM((DEG_CHUNKS, CHUNK), jnp.int32),
        pltpu.VMEM((CHUNK, 128), jnp.float32),
        pltpu.VMEM((CHUNK, 128), jnp.float32),
        pltpu.VMEM_SHARED((NPAD, 128), jnp.float32),
    ],
)
def _deg_kernel(dst_hbm, out_hbm, idx_v, ones_v, stage_v, acc):
    c = lax.axis_index("c")
    s = lax.axis_index("s")
    w = c * NS + s

    def fill(i, _):
        for q in range(128 // 16):
            ones_v[i, pl.ds(q * 16, 16)] = jnp.full((16,), 1.0, jnp.float32)
        return 0
    lax.fori_loop(0, CHUNK, fill, 0)

    _zero_stage(stage_v)
    _zero_acc_stripe(s, stage_v, acc)
    plsc.subcore_barrier()

    for t in range(NC * NS):
        @pl.when(w == t)
        def _():
            pltpu.sync_copy(dst_hbm.at[t], idx_v)

    def body(j, _):
        pltpu.sync_copy(ones_v, acc.at[idx_v.at[j]], add=True)
        return 0
    lax.fori_loop(0, DEG_CHUNKS, body, 0)
    plsc.subcore_barrier()

    _flush_acc_stripe(c, s, stage_v, acc, out_hbm)


@functools.partial(
    pl.kernel,
    out_type=jax.ShapeDtypeStruct((NC * NPAD, H), jnp.float32),
    mesh=_mesh,
    scratch_types=[
        pltpu.VMEM((SEG_CHUNKS, CHUNK), jnp.int32),
        pltpu.VMEM((SEG_CHUNKS, CHUNK), jnp.int32),
        pltpu.VMEM((CHUNK, H), jnp.float32),
        pltpu.VMEM_SHARED((NPAD, H), jnp.float32),
        pltpu.SemaphoreType.DMA,
    ],
)
def _segsum_kernel(glo_hbm, ghi_hbm, src_hbm, dst_hbm, out_hbm,
                   src_v, dst_v, rows_v, acc, sem):
    c = lax.axis_index("c")
    s = lax.axis_index("s")

    _zero_stage(rows_v)
    _zero_acc_stripe(s, rows_v, acc)
    plsc.subcore_barrier()

    for t in range(NS):
        @pl.when(s == t)
        def _():
            pltpu.sync_copy(src_hbm.at[t], src_v)
            pltpu.sync_copy(dst_hbm.at[t], dst_v)

    for cc, table in ((0, glo_hbm), (1, ghi_hbm)):
        @pl.when(c == cc)
        def _():
            def body(j, _):
                pltpu.async_copy(table.at[src_v.at[j]], rows_v, sem).wait()
                pltpu.sync_copy(rows_v, acc.at[dst_v.at[j]], add=True)
                return 0
            lax.fori_loop(0, SEG_CHUNKS, body, 0)
    plsc.subcore_barrier()

    _flush_acc_stripe(c, s, rows_v, acc, out_hbm)


RB = 1000


def _dinv_of(deg2):
    return lax.rsqrt(jnp.sum(deg2, axis=(0, 2)) * (1.0 / 128.0) + 1.0)


def _tc1_body(deg2_ref, x_ref, w_ref, glo_ref, ghi_ref):
    dinv = _dinv_of(deg2_ref[...])
    h = jnp.dot(x_ref[...], w_ref[...], preferred_element_type=jnp.float32)
    g = h * dinv[:, None]
    glo_ref[...] = g[:, :H]
    ghi_ref[...] = g[:, H:]


def _tc2_body(deg2_ref, s_ref, glo_ref, ghi_ref, w_ref, b_ref,
              olo_ref, ohi_ref):
    dinv = _dinv_of(deg2_ref[...])
    t = jnp.concatenate(
        [s_ref[0] + glo_ref[...], s_ref[1] + ghi_ref[...]], axis=1)
    z = jax.nn.relu(dinv[:, None] * t + b_ref[0][None, :])
    h = jnp.dot(z, w_ref[...], preferred_element_type=jnp.float32)
    g = h * dinv[:, None]
    olo_ref[...] = g[:, :H]
    ohi_ref[...] = g[:, H:]


def _tc3_body(deg2_ref, s_ref, glo_ref, ghi_ref, b_ref, o_ref):
    dinv = _dinv_of(deg2_ref[...])
    t = jnp.concatenate(
        [s_ref[0] + glo_ref[...], s_ref[1] + ghi_ref[...]], axis=1)
    o_ref[...] = dinv[:, None] * t + b_ref[0][None, :]


_deg2_spec = pl.BlockSpec((NC, RB, 128), lambda i: (0, i, 0))
_s_spec = pl.BlockSpec((NC, RB, H), lambda i: (0, i, 0))
_g_spec = pl.BlockSpec((RB, H), lambda i: (i, 0))
_b_spec = pl.BlockSpec((1, D), lambda i: (0, 0))
_w_spec = pl.BlockSpec((D, D), lambda i: (0, 0))
_gout = jax.ShapeDtypeStruct((N, H), jnp.float32)

_tc1 = pl.pallas_call(
    _tc1_body,
    grid=(N // RB,),
    in_specs=[_deg2_spec, pl.BlockSpec((RB, D), lambda i: (i, 0)), _w_spec],
    out_specs=(_g_spec, _g_spec),
    out_shape=(_gout, _gout),
)

_tc2 = pl.pallas_call(
    _tc2_body,
    grid=(N // RB,),
    in_specs=[_deg2_spec, _s_spec, _g_spec, _g_spec, _w_spec, _b_spec],
    out_specs=(_g_spec, _g_spec),
    out_shape=(_gout, _gout),
)

_tc3 = pl.pallas_call(
    _tc3_body,
    grid=(N // RB,),
    in_specs=[_deg2_spec, _s_spec, _g_spec, _g_spec, _b_spec],
    out_specs=pl.BlockSpec((RB, D), lambda i: (i, 0)),
    out_shape=jax.ShapeDtypeStruct((N, D), jnp.float32),
)


def kernel(x, edge_index, W1, b1, W2, b2):
    src = edge_index[0].astype(jnp.int32)
    dst = edge_index[1].astype(jnp.int32)
    pad = EPAD - E
    src = jnp.concatenate([src, jnp.zeros((pad,), jnp.int32)])
    dst = jnp.concatenate([dst, jnp.full((pad,), N, jnp.int32)])
    dst_deg = dst.reshape(NC * NS, DEG_CHUNKS, CHUNK)
    src_seg = src.reshape(NS, SEG_CHUNKS, CHUNK)
    dst_seg = dst.reshape(NS, SEG_CHUNKS, CHUNK)

    deg2 = _deg_kernel(dst_deg).reshape(NC, NPAD, 128)

    g1lo, g1hi = _tc1(deg2, x, W1)
    s1 = _segsum_kernel(g1lo, g1hi, src_seg, dst_seg).reshape(NC, NPAD, H)
    g2lo, g2hi = _tc2(deg2, s1, g1lo, g1hi, W2, jnp.reshape(b1, (1, D)))
    s2 = _segsum_kernel(g2lo, g2hi, src_seg, dst_seg).reshape(NC, NPAD, H)
    return _tc3(deg2, s2, g2lo, g2hi, jnp.reshape(b2, (1, D)))


# speedup vs baseline: 7.4500x; 1.1135x over previous
import functools

import jax
import jax.numpy as jnp
from jax import lax
from jax.experimental import pallas as pl
from jax.experimental.pallas import tpu as pltpu
from jax.experimental.pallas import tpu_sc as plsc

N = 10000
E = 160000
D = 256
H = D // 2
NC = 2
NS = 16
CHUNK = 128

NPAD = 10240
EPAD = 163840
ROWS_PER_TILE = NPAD // NS
SEG_CHUNKS = EPAD // (NS * CHUNK)
DEG_CHUNKS = EPAD // (NC * NS * CHUNK)

_mesh = plsc.VectorSubcoreMesh(core_axis_name="c", subcore_axis_name="s")


def _zero_stage(stage_v):
    def zfill(i, _):
        for q in range(128 // 16):
            stage_v[i, pl.ds(q * 16, 16)] = jnp.zeros((16,), jnp.float32)
        return 0
    lax.fori_loop(0, CHUNK, zfill, 0)


def _zero_acc_stripe(s, stage_v, acc):
    for t in range(NS):
        @pl.when(s == t)
        def _():
            for j in range(ROWS_PER_TILE // CHUNK):
                pltpu.sync_copy(
                    stage_v, acc.at[pl.ds(t * ROWS_PER_TILE + j * CHUNK, CHUNK)])


def _flush_acc_stripe(c, s, stage_v, acc, out_hbm):
    for t in range(NS):
        @pl.when(s == t)
        def _():
            for j in range(ROWS_PER_TILE // CHUNK):
                off = t * ROWS_PER_TILE + j * CHUNK
                pltpu.sync_copy(acc.at[pl.ds(off, CHUNK)], stage_v)
                pltpu.sync_copy(stage_v, out_hbm.at[pl.ds(c * NPAD + off, CHUNK)])


@functools.partial(
    pl.kernel,
    out_type=jax.ShapeDtypeStruct((NC * NPAD, 128), jnp.float32),
    mesh=_mesh,
    scratch_types=[
        pltpu.VMEM((DEG_CHUNKS, CHUNK), jnp.int32),
        pltpu.VMEM((CHUNK, 128), jnp.float32),
        pltpu.VMEM((CHUNK, 128), jnp.float32),
        pltpu.VMEM_SHARED((NPAD, 128), jnp.float32),
    ],
)
def _deg_kernel(dst_hbm, out_hbm, idx_v, ones_v, stage_v, acc):
    c = lax.axis_index("c")
    s = lax.axis_index("s")
    w = c * NS + s

    def fill(i, _):
        for q in range(128 // 16):
            ones_v[i, pl.ds(q * 16, 16)] = jnp.full((16,), 1.0, jnp.float32)
        return 0
    lax.fori_loop(0, CHUNK, fill, 0)

    _zero_stage(stage_v)
    _zero_acc_stripe(s, stage_v, acc)
    plsc.subcore_barrier()

    for t in range(NC * NS):
        @pl.when(w == t)
        def _():
            pltpu.sync_copy(dst_hbm.at[t], idx_v)

    def body(j, _):
        pltpu.sync_copy(ones_v, acc.at[idx_v.at[j]], add=True)
        return 0
    lax.fori_loop(0, DEG_CHUNKS, body, 0)
    plsc.subcore_barrier()

    _flush_acc_stripe(c, s, stage_v, acc, out_hbm)


HALVES = 2
HALF_CHUNKS = SEG_CHUNKS // HALVES
PAIRS = HALF_CHUNKS // 2


@functools.partial(
    pl.kernel,
    out_type=jax.ShapeDtypeStruct((NC * NPAD, H), jnp.float32),
    mesh=_mesh,
    scratch_types=[
        pltpu.VMEM((HALF_CHUNKS, CHUNK), jnp.int32),
        pltpu.VMEM((HALF_CHUNKS, CHUNK), jnp.int32),
        pltpu.VMEM((CHUNK, H), jnp.float32),
        pltpu.VMEM((CHUNK, H), jnp.float32),
        pltpu.VMEM_SHARED((NPAD, H), jnp.float32),
        pltpu.SemaphoreType.DMA,
        pltpu.SemaphoreType.DMA,
        pltpu.SemaphoreType.DMA,
        pltpu.SemaphoreType.DMA,
    ],
)
def _segsum_kernel(glo_hbm, ghi_hbm, src_hbm, dst_hbm, out_hbm,
                   src_v, dst_v, rows_a, rows_b, acc,
                   sem_ga, sem_gb, sem_sa, sem_sb):
    c = lax.axis_index("c")
    s = lax.axis_index("s")

    _zero_stage(rows_a)
    _zero_acc_stripe(s, rows_a, acc)
    plsc.subcore_barrier()

    for cc, table in ((0, glo_hbm), (1, ghi_hbm)):
        @pl.when(c == cc)
        def _(table=table):
            for half in range(HALVES):
                for t in range(NS):
                    @pl.when(s == t)
                    def _(t=t):
                        pltpu.sync_copy(src_hbm.at[t, half], src_v)
                        pltpu.sync_copy(dst_hbm.at[t, half], dst_v)

                pltpu.async_copy(table.at[src_v.at[0]], rows_a, sem_ga)

                def body(j, _, table=table):
                    a = 2 * j
                    b = a + 1
                    ga = pltpu.make_async_copy(
                        table.at[src_v.at[a]], rows_a, sem_ga)
                    gb = pltpu.make_async_copy(
                        table.at[src_v.at[b]], rows_b, sem_gb)
                    sa = pltpu.make_async_copy(
                        rows_a, acc.at[dst_v.at[a]], sem_sa)
                    sb = pltpu.make_async_copy(
                        rows_b, acc.at[dst_v.at[b]], sem_sb)
                    ga.wait()
                    pltpu.async_copy(table.at[src_v.at[b]], rows_b, sem_gb)
                    pltpu.async_copy(
                        rows_a, acc.at[dst_v.at[a]], sem_sa, add=True)
                    gb.wait()
                    pltpu.async_copy(
                        rows_b, acc.at[dst_v.at[b]], sem_sb, add=True)
                    sa.wait()

                    @pl.when(j < PAIRS - 1)
                    def _(table=table):
                        pltpu.async_copy(
                            table.at[src_v.at[a + 2]], rows_a, sem_ga)
                    sb.wait()
                    return 0
                lax.fori_loop(0, PAIRS, body, 0)
    plsc.subcore_barrier()

    _flush_acc_stripe(c, s, rows_a, acc, out_hbm)


RB = 1000


def _dinv_of(deg2):
    return lax.rsqrt(jnp.sum(deg2, axis=(0, 2)) * (1.0 / 128.0) + 1.0)


def _tc1_body(deg2_ref, x_ref, w_ref, glo_ref, ghi_ref):
    dinv = _dinv_of(deg2_ref[...])
    h = jnp.dot(x_ref[...], w_ref[...], preferred_element_type=jnp.float32)
    g = h * dinv[:, None]
    glo_ref[...] = g[:, :H]
    ghi_ref[...] = g[:, H:]


def _tc2_body(deg2_ref, s_ref, glo_ref, ghi_ref, w_ref, b_ref,
              olo_ref, ohi_ref):
    dinv = _dinv_of(deg2_ref[...])
    t = jnp.concatenate(
        [s_ref[0] + glo_ref[...], s_ref[1] + ghi_ref[...]], axis=1)
    z = jax.nn.relu(dinv[:, None] * t + b_ref[0][None, :])
    h = jnp.dot(z, w_ref[...], preferred_element_type=jnp.float32)
    g = h * dinv[:, None]
    olo_ref[...] = g[:, :H]
    ohi_ref[...] = g[:, H:]


def _tc3_body(deg2_ref, s_ref, glo_ref, ghi_ref, b_ref, o_ref):
    dinv = _dinv_of(deg2_ref[...])
    t = jnp.concatenate(
        [s_ref[0] + glo_ref[...], s_ref[1] + ghi_ref[...]], axis=1)
    o_ref[...] = dinv[:, None] * t + b_ref[0][None, :]


_deg2_spec = pl.BlockSpec((NC, RB, 128), lambda i: (0, i, 0))
_s_spec = pl.BlockSpec((NC, RB, H), lambda i: (0, i, 0))
_g_spec = pl.BlockSpec((RB, H), lambda i: (i, 0))
_b_spec = pl.BlockSpec((1, D), lambda i: (0, 0))
_w_spec = pl.BlockSpec((D, D), lambda i: (0, 0))
_gout = jax.ShapeDtypeStruct((N, H), jnp.float32)

_tc1 = pl.pallas_call(
    _tc1_body,
    grid=(N // RB,),
    in_specs=[_deg2_spec, pl.BlockSpec((RB, D), lambda i: (i, 0)), _w_spec],
    out_specs=(_g_spec, _g_spec),
    out_shape=(_gout, _gout),
)

_tc2 = pl.pallas_call(
    _tc2_body,
    grid=(N // RB,),
    in_specs=[_deg2_spec, _s_spec, _g_spec, _g_spec, _w_spec, _b_spec],
    out_specs=(_g_spec, _g_spec),
    out_shape=(_gout, _gout),
)

_tc3 = pl.pallas_call(
    _tc3_body,
    grid=(N // RB,),
    in_specs=[_deg2_spec, _s_spec, _g_spec, _g_spec, _b_spec],
    out_specs=pl.BlockSpec((RB, D), lambda i: (i, 0)),
    out_shape=jax.ShapeDtypeStruct((N, D), jnp.float32),
)


def kernel(x, edge_index, W1, b1, W2, b2):
    src = edge_index[0].astype(jnp.int32)
    dst = edge_index[1].astype(jnp.int32)
    pad = EPAD - E
    src = jnp.concatenate([src, jnp.zeros((pad,), jnp.int32)])
    dst = jnp.concatenate([dst, jnp.full((pad,), N, jnp.int32)])
    dst_deg = dst.reshape(NC * NS, DEG_CHUNKS, CHUNK)
    src_seg = src.reshape(NS, HALVES, HALF_CHUNKS, CHUNK)
    dst_seg = dst.reshape(NS, HALVES, HALF_CHUNKS, CHUNK)

    deg2 = _deg_kernel(dst_deg).reshape(NC, NPAD, 128)

    g1lo, g1hi = _tc1(deg2, x, W1)
    s1 = _segsum_kernel(g1lo, g1hi, src_seg, dst_seg).reshape(NC, NPAD, H)
    g2lo, g2hi = _tc2(deg2, s1, g1lo, g1hi, W2, jnp.reshape(b1, (1, D)))
    s2 = _segsum_kernel(g2lo, g2hi, src_seg, dst_seg).reshape(NC, NPAD, H)
    return _tc3(deg2, s2, g2lo, g2hi, jnp.reshape(b2, (1, D)))


# speedup vs baseline: 7.9060x; 1.0612x over previous
import functools

import jax
import jax.numpy as jnp
from jax import lax
from jax.experimental import pallas as pl
from jax.experimental.pallas import tpu as pltpu
from jax.experimental.pallas import tpu_sc as plsc

N = 10000
E = 160000
D = 256
H = D // 2
NC = 2
NS = 16
CHUNK = 128

NPAD = 10240
EPAD = 163840
ROWS_PER_TILE = NPAD // NS
SEG_CHUNKS = EPAD // (NS * CHUNK)
DEG_CHUNKS = EPAD // (NC * NS * CHUNK)

_mesh = plsc.VectorSubcoreMesh(core_axis_name="c", subcore_axis_name="s")


def _zero_stage(stage_v):
    def zfill(i, _):
        for q in range(128 // 16):
            stage_v[i, pl.ds(q * 16, 16)] = jnp.zeros((16,), jnp.float32)
        return 0
    lax.fori_loop(0, CHUNK, zfill, 0)


def _zero_acc_stripe(s, stage_v, acc):
    for t in range(NS):
        @pl.when(s == t)
        def _():
            for j in range(ROWS_PER_TILE // CHUNK):
                pltpu.sync_copy(
                    stage_v, acc.at[pl.ds(t * ROWS_PER_TILE + j * CHUNK, CHUNK)])


def _flush_acc_stripe(c, s, stage_v, acc, out_hbm):
    for t in range(NS):
        @pl.when(s == t)
        def _():
            for j in range(ROWS_PER_TILE // CHUNK):
                off = t * ROWS_PER_TILE + j * CHUNK
                pltpu.sync_copy(acc.at[pl.ds(off, CHUNK)], stage_v)
                pltpu.sync_copy(stage_v, out_hbm.at[pl.ds(c * NPAD + off, CHUNK)])


@functools.partial(
    pl.kernel,
    out_type=jax.ShapeDtypeStruct((NC * NPAD, 128), jnp.float32),
    mesh=_mesh,
    scratch_types=[
        pltpu.VMEM((DEG_CHUNKS, CHUNK), jnp.int32),
        pltpu.VMEM((CHUNK, 128), jnp.float32),
        pltpu.VMEM((CHUNK, 128), jnp.float32),
        pltpu.VMEM_SHARED((NPAD, 128), jnp.float32),
    ],
)
def _deg_kernel(dst_hbm, out_hbm, idx_v, ones_v, stage_v, acc):
    c = lax.axis_index("c")
    s = lax.axis_index("s")
    w = c * NS + s

    def fill(i, _):
        for q in range(128 // 16):
            ones_v[i, pl.ds(q * 16, 16)] = jnp.full((16,), 1.0, jnp.float32)
        return 0
    lax.fori_loop(0, CHUNK, fill, 0)

    _zero_stage(stage_v)
    _zero_acc_stripe(s, stage_v, acc)
    plsc.subcore_barrier()

    for t in range(NC * NS):
        @pl.when(w == t)
        def _():
            pltpu.sync_copy(dst_hbm.at[t], idx_v)

    def body(j, _):
        pltpu.sync_copy(ones_v, acc.at[idx_v.at[j]], add=True)
        return 0
    lax.fori_loop(0, DEG_CHUNKS, body, 0)
    plsc.subcore_barrier()

    _flush_acc_stripe(c, s, stage_v, acc, out_hbm)


HALVES = 2
HALF_CHUNKS = SEG_CHUNKS // HALVES
PAIRS = HALF_CHUNKS // 2


@functools.partial(
    pl.kernel,
    out_type=jax.ShapeDtypeStruct((NC * NPAD, H), jnp.float32),
    mesh=_mesh,
    scratch_types=[
        pltpu.VMEM((HALF_CHUNKS, CHUNK), jnp.int32),
        pltpu.VMEM((HALF_CHUNKS, CHUNK), jnp.int32),
        pltpu.VMEM((CHUNK, H), jnp.float32),
        pltpu.VMEM((CHUNK, H), jnp.float32),
        pltpu.VMEM_SHARED((NPAD, H), jnp.float32),
        pltpu.SemaphoreType.DMA,
        pltpu.SemaphoreType.DMA,
        pltpu.SemaphoreType.DMA,
        pltpu.SemaphoreType.DMA,
    ],
)
def _segsum_kernel(glo_hbm, ghi_hbm, src_hbm, dst_hbm, out_hbm,
                   src_v, dst_v, rows_a, rows_b, acc,
                   sem_ga, sem_gb, sem_sa, sem_sb):
    c = lax.axis_index("c")
    s = lax.axis_index("s")

    _zero_stage(rows_a)
    _zero_acc_stripe(s, rows_a, acc)
    plsc.subcore_barrier()

    for cc, table in ((0, glo_hbm), (1, ghi_hbm)):
        @pl.when(c == cc)
        def _(table=table):
            for half in range(HALVES):
                for t in range(NS):
                    @pl.when(s == t)
                    def _(t=t):
                        pltpu.sync_copy(src_hbm.at[t, half], src_v)
                        pltpu.sync_copy(dst_hbm.at[t, half], dst_v)

                pltpu.async_copy(table.at[src_v.at[0]], rows_a, sem_ga)

                def body(j, _, table=table):
                    a = 2 * j
                    b = a + 1
                    ga = pltpu.make_async_copy(
                        table.at[src_v.at[a]], rows_a, sem_ga)
                    gb = pltpu.make_async_copy(
                        table.at[src_v.at[b]], rows_b, sem_gb)
                    sa = pltpu.make_async_copy(
                        rows_a, acc.at[dst_v.at[a]], sem_sa)
                    sb = pltpu.make_async_copy(
                        rows_b, acc.at[dst_v.at[b]], sem_sb)
                    ga.wait()
                    pltpu.async_copy(table.at[src_v.at[b]], rows_b, sem_gb)
                    pltpu.async_copy(
                        rows_a, acc.at[dst_v.at[a]], sem_sa, add=True)
                    gb.wait()
                    pltpu.async_copy(
                        rows_b, acc.at[dst_v.at[b]], sem_sb, add=True)
                    sa.wait()

                    @pl.when(j < PAIRS - 1)
                    def _(table=table):
                        pltpu.async_copy(
                            table.at[src_v.at[a + 2]], rows_a, sem_ga)
                    sb.wait()
                    return 0
                lax.fori_loop(0, PAIRS, body, 0)
    plsc.subcore_barrier()

    _flush_acc_stripe(c, s, rows_a, acc, out_hbm)


RB = 1000


def _dinv_of(deg2):
    return lax.rsqrt(jnp.sum(deg2, axis=(0, 2)) * (1.0 / 128.0) + 1.0)


def _tc1a_body(x_ref, w_ref, h_ref):
    h_ref[...] = jnp.dot(
        x_ref[...], w_ref[...], preferred_element_type=jnp.float32)


def _tc1b_body(deg2_ref, h_ref, glo_ref, ghi_ref, dinv_ref):
    dinv = _dinv_of(deg2_ref[...])
    g = h_ref[...] * dinv[:, None]
    glo_ref[...] = g[:, :H]
    ghi_ref[...] = g[:, H:]
    dinv_ref[...] = jnp.broadcast_to(dinv[:, None], (RB, 128))


def _tc2_body(dinv_ref, s_ref, glo_ref, ghi_ref, w_ref, b_ref,
              olo_ref, ohi_ref):
    dinv = dinv_ref[...][:, :1]
    t = jnp.concatenate(
        [s_ref[0] + glo_ref[...], s_ref[1] + ghi_ref[...]], axis=1)
    z = jax.nn.relu(dinv * t + b_ref[0][None, :])
    h = jnp.dot(z, w_ref[...], preferred_element_type=jnp.float32)
    g = h * dinv
    olo_ref[...] = g[:, :H]
    ohi_ref[...] = g[:, H:]


def _tc3_body(dinv_ref, s_ref, glo_ref, ghi_ref, b_ref, o_ref):
    dinv = dinv_ref[...][:, :1]
    t = jnp.concatenate(
        [s_ref[0] + glo_ref[...], s_ref[1] + ghi_ref[...]], axis=1)
    o_ref[...] = dinv * t + b_ref[0][None, :]


_deg2_spec = pl.BlockSpec((NC, RB, 128), lambda i: (0, i, 0))
_s_spec = pl.BlockSpec((NC, RB, H), lambda i: (0, i, 0))
_g_spec = pl.BlockSpec((RB, H), lambda i: (i, 0))
_b_spec = pl.BlockSpec((1, D), lambda i: (0, 0))
_w_spec = pl.BlockSpec((D, D), lambda i: (0, 0))
_gout = jax.ShapeDtypeStruct((N, H), jnp.float32)

_x_spec = pl.BlockSpec((RB, D), lambda i: (i, 0))
_dinv_spec = pl.BlockSpec((RB, 128), lambda i: (i, 0))

_tc1a = pl.pallas_call(
    _tc1a_body,
    grid=(N // RB,),
    in_specs=[_x_spec, _w_spec],
    out_specs=_x_spec,
    out_shape=jax.ShapeDtypeStruct((N, D), jnp.float32),
)

_tc1b = pl.pallas_call(
    _tc1b_body,
    grid=(N // RB,),
    in_specs=[_deg2_spec, _x_spec],
    out_specs=(_g_spec, _g_spec, _dinv_spec),
    out_shape=(_gout, _gout, jax.ShapeDtypeStruct((N, 128), jnp.float32)),
)

_tc2 = pl.pallas_call(
    _tc2_body,
    grid=(N // RB,),
    in_specs=[_dinv_spec, _s_spec, _g_spec, _g_spec, _w_spec, _b_spec],
    out_specs=(_g_spec, _g_spec),
    out_shape=(_gout, _gout),
)

_tc3 = pl.pallas_call(
    _tc3_body,
    grid=(N // RB,),
    in_specs=[_dinv_spec, _s_spec, _g_spec, _g_spec, _b_spec],
    out_specs=pl.BlockSpec((RB, D), lambda i: (i, 0)),
    out_shape=jax.ShapeDtypeStruct((N, D), jnp.float32),
)


def kernel(x, edge_index, W1, b1, W2, b2):
    src = edge_index[0].astype(jnp.int32)
    dst = edge_index[1].astype(jnp.int32)
    pad = EPAD - E
    src = jnp.concatenate([src, jnp.zeros((pad,), jnp.int32)])
    dst = jnp.concatenate([dst, jnp.full((pad,), N, jnp.int32)])
    dst_deg = dst.reshape(NC * NS, DEG_CHUNKS, CHUNK)
    src_seg = src.reshape(NS, HALVES, HALF_CHUNKS, CHUNK)
    dst_seg = dst.reshape(NS, HALVES, HALF_CHUNKS, CHUNK)

    deg2 = _deg_kernel(dst_deg).reshape(NC, NPAD, 128)
    h1 = _tc1a(x, W1)

    g1lo, g1hi, dinvb = _tc1b(deg2, h1)
    s1 = _segsum_kernel(g1lo, g1hi, src_seg, dst_seg).reshape(NC, NPAD, H)
    g2lo, g2hi = _tc2(dinvb, s1, g1lo, g1hi, W2, jnp.reshape(b1, (1, D)))
    s2 = _segsum_kernel(g2lo, g2hi, src_seg, dst_seg).reshape(NC, NPAD, H)
    return _tc3(dinvb, s2, g2lo, g2hi, jnp.reshape(b2, (1, D)))
